# Initial kernel scaffold; baseline (speedup 1.0000x reference)
import jax
import jax.numpy as jnp
from jax.experimental import pallas as pl


def kernel(inputs, indices, axis):
    raise NotImplementedError("write your pallas kernel here")


# speedup vs baseline: 3.3148x; 3.3148x over previous
import functools

import jax
import jax.numpy as jnp
from jax import lax
from jax.experimental import pallas as pl
from jax.experimental.pallas import tpu as pltpu
from jax.experimental.pallas import tpu_sc as plsc

_NUM_CORES = 2
_NUM_SUBCORES = 16
_NW = _NUM_CORES * _NUM_SUBCORES
_C = 128


@functools.lru_cache(maxsize=None)
def _build_gather(V, D, B):
  assert B % (_NW * _C) == 0
  b_per_w = B // _NW
  n_chunks = b_per_w // _C
  mesh = plsc.VectorSubcoreMesh(core_axis_name="c", subcore_axis_name="s")

  @functools.partial(
      pl.kernel,
      out_type=jax.ShapeDtypeStruct((B, D), jnp.float32),
      mesh=mesh,
      scratch_types=[
          pltpu.VMEM((n_chunks, _C), jnp.int32),
          pltpu.VMEM((2, _C, D), jnp.float32),
          pltpu.SemaphoreType.DMA,
          pltpu.SemaphoreType.DMA,
          pltpu.SemaphoreType.DMA,
          pltpu.SemaphoreType.DMA,
      ],
  )
  def gather_kernel(table_hbm, idx_hbm, out_hbm, idx_v, rows_v, g0, g1, o0, o1):
    gsems = (g0, g1)
    osems = (o0, o1)
    wid = lax.axis_index("s") * _NUM_CORES + lax.axis_index("c")
    base = wid * b_per_w

    pltpu.sync_copy(idx_hbm.at[wid], idx_v)

    def start_gather(j, slot):
      pltpu.async_copy(table_hbm.at[idx_v.at[j]], rows_v.at[slot], gsems[slot])

    def wait_gather(j, slot):
      pltpu.make_async_copy(
          table_hbm.at[idx_v.at[j]], rows_v.at[slot], gsems[slot]).wait()

    def out_write(j, slot):
      return pltpu.make_async_copy(
          rows_v.at[slot], out_hbm.at[pl.ds(base + j * _C, _C)], osems[slot])

    for b in range(2):
      start_gather(b, b)

    @pl.loop(0, n_chunks // 2 - 1)
    def _(jo):
      for b in range(2):
        j = 2 * jo + b
        wait_gather(j, b)
        out_write(j, b).start()
        out_write(j, b).wait()
        start_gather(j + 2, b)

    for b in range(2):
      j = n_chunks - 2 + b
      wait_gather(j, b)
      out_write(j, b).start()
      out_write(j, b).wait()

  return gather_kernel


def kernel(inputs, indices, axis):
  del axis
  V, D = inputs.shape
  B = indices.size
  idx_mat = indices.astype(jnp.int32).reshape(_NW, B // (_NW * _C), _C)
  out = _build_gather(V, D, B)(inputs, idx_mat)
  return out.reshape(indices.shape + (D,))


# speedup vs baseline: 5.1321x; 1.5483x over previous
import functools

import jax
import jax.numpy as jnp
from jax import lax
from jax.experimental import pallas as pl
from jax.experimental.pallas import tpu as pltpu
from jax.experimental.pallas import tpu_sc as plsc

_NUM_CORES = 2
_NUM_SUBCORES = 16
_NW = _NUM_CORES * _NUM_SUBCORES


@functools.lru_cache(maxsize=None)
def _build_gather(V, D, N, K):
  assert N % (_NW * 2) == 0 and K <= 128
  n_per_w = N // _NW
  mesh = plsc.VectorSubcoreMesh(core_axis_name="c", subcore_axis_name="s")

  @functools.partial(
      pl.kernel,
      out_type=jax.ShapeDtypeStruct((N, K, D), jnp.float32),
      mesh=mesh,
      scratch_types=[
          pltpu.VMEM((n_per_w, K), jnp.int32),
          pltpu.VMEM((2, K, D), jnp.float32),
          pltpu.SemaphoreType.DMA,
          pltpu.SemaphoreType.DMA,
          pltpu.SemaphoreType.DMA,
          pltpu.SemaphoreType.DMA,
      ],
  )
  def gather_kernel(table_hbm, idx_hbm, out_hbm, idx_v, rows_v, g0, g1, o0, o1):
    gsems = (g0, g1)
    osems = (o0, o1)
    wid = lax.axis_index("s") * _NUM_CORES + lax.axis_index("c")
    base = wid * n_per_w

    pltpu.sync_copy(idx_hbm.at[pl.ds(base, n_per_w)], idx_v)

    def start_gather(j, slot):
      pltpu.async_copy(table_hbm.at[idx_v.at[j]], rows_v.at[slot], gsems[slot])

    def wait_gather(j, slot):
      pltpu.make_async_copy(
          table_hbm.at[idx_v.at[j]], rows_v.at[slot], gsems[slot]).wait()

    def out_write(j, slot):
      return pltpu.make_async_copy(
          rows_v.at[slot], out_hbm.at[base + j], osems[slot])

    for b in range(2):
      start_gather(b, b)

    @pl.loop(0, n_per_w // 2 - 1)
    def _(jo):
      for b in range(2):
        j = 2 * jo + b
        wait_gather(j, b)
        out_write(j, b).start()
        out_write(j, b).wait()
        start_gather(j + 2, b)

    for b in range(2):
      j = n_per_w - 2 + b
      wait_gather(j, b)
      out_write(j, b).start()
      out_write(j, b).wait()

  return gather_kernel


def kernel(inputs, indices, axis):
  del axis
  V, D = inputs.shape
  N, K = indices.shape
  return _build_gather(V, D, N, K)(inputs, indices.astype(jnp.int32))


# speedup vs baseline: 5.9602x; 1.1614x over previous
import functools

import jax
import jax.numpy as jnp
from jax import lax
from jax.experimental import pallas as pl
from jax.experimental.pallas import tpu as pltpu
from jax.experimental.pallas import tpu_sc as plsc

_NUM_CORES = 2
_NUM_SUBCORES = 16
_NW = _NUM_CORES * _NUM_SUBCORES
_L = 4


@functools.lru_cache(maxsize=None)
def _build_gather(V, D, N, K):
  nb = 2 * _L
  n_per_w = N // _NW
  assert N % _NW == 0 and K <= 128
  assert n_per_w % nb == 0 and n_per_w >= 2 * nb
  mesh = plsc.VectorSubcoreMesh(core_axis_name="c", subcore_axis_name="s")

  @functools.partial(
      pl.kernel,
      out_type=jax.ShapeDtypeStruct((N, K, D), jnp.float32),
      mesh=mesh,
      scratch_types=[
          pltpu.VMEM((n_per_w, K), jnp.int32),
          pltpu.VMEM((nb, K, D), jnp.float32),
          [pltpu.SemaphoreType.DMA] * nb,
          [pltpu.SemaphoreType.DMA] * nb,
      ],
  )
  def gather_kernel(table_hbm, idx_hbm, out_hbm, idx_v, rows_v, gsems, osems):
    wid = lax.axis_index("s") * _NUM_CORES + lax.axis_index("c")
    base = wid * n_per_w

    pltpu.sync_copy(idx_hbm.at[pl.ds(base, n_per_w)], idx_v)

    def gather(j, slot):
      return pltpu.make_async_copy(
          table_hbm.at[idx_v.at[j]], rows_v.at[slot], gsems[slot])

    def out_write(j, slot):
      return pltpu.make_async_copy(
          rows_v.at[slot], out_hbm.at[base + j], osems[slot])

    for s in range(_L):
      gather(s, s).start()
    for j in range(_L):
      gather(j + _L, j + _L).start()
      gather(j, j).wait()
      out_write(j, j).start()

    @pl.loop(0, (n_per_w - 2 * _L) // nb)
    def _(jo):
      for b in range(nb):
        j = _L + nb * jo + b
        sn = (b + 2 * _L) % nb
        out_write(j - _L, sn).wait()
        gather(j + _L, sn).start()
        s = (_L + b) % nb
        gather(j, s).wait()
        out_write(j, s).start()

    for j in range(n_per_w - _L, n_per_w):
      s = j % nb
      gather(j, s).wait()
      out_write(j, s).start()
    for j in range(n_per_w - nb, n_per_w):
      out_write(j, j % nb).wait()

  return gather_kernel


def kernel(inputs, indices, axis):
  del axis
  V, D = inputs.shape
  N, K = indices.shape
  return _build_gather(V, D, N, K)(inputs, indices.astype(jnp.int32))
